# baseline (speedup 1.0000x reference)
import jax
import jax.numpy as jnp
import numpy as np
from jax.experimental import pallas as pl
from jax.experimental.pallas import tpu as pltpu

N_NODES = 10000
N_GRAPHS = 8
HEADS = 8
HID = 128
HD = HID // HEADS


def _apply(p, x):
    return x @ p[0] + p[1]


def _leaky(x):
    return jax.nn.leaky_relu(x, 0.2)


def _bn(x):
    return (x - x.mean(0)) / jnp.sqrt(x.var(0) + 1e-5)


def _gine(p, x, src, dst, ea):
    msg = jax.nn.relu(x[src] + ea)
    agg = jax.ops.segment_sum(msg, dst, num_segments=N_NODES)
    h = (1.0 + p["eps"]) * x + agg
    h = jax.nn.relu(_apply(p["l1"], h))
    h = _apply(p["l2"], h)
    return jax.nn.relu(h)


def _tconv(p, x, src, dst, ea):
    q = _apply(p["q"], x).reshape(N_NODES, HEADS, HD)
    k = _apply(p["k"], x).reshape(N_NODES, HEADS, HD)
    v = _apply(p["v"], x).reshape(N_NODES, HEADS, HD)
    e = (ea @ p["We"]).reshape(-1, HEADS, HD)
    kj = k[src] + e
    alpha = (q[dst] * kj).sum(-1) / np.sqrt(HD)
    amax = jax.ops.segment_max(alpha, dst, num_segments=N_NODES)
    amax = jnp.where(jnp.isfinite(amax), amax, 0.0)
    al = jnp.exp(alpha - amax[dst])
    den = jax.ops.segment_sum(al, dst, num_segments=N_NODES)
    al = al / (den[dst] + 1e-16)
    msg = (v[src] + e) * al[..., None]
    out = jax.ops.segment_sum(msg, dst, num_segments=N_NODES).reshape(N_NODES, HID)
    return out + _apply(p["s"], x)


def _fc_kernel(pooled_ref, w0, b0, w1, b1, w2, b2, w3, b3, out_ref):
    o = jnp.maximum(pooled_ref[...] @ w0[...] + b0[...], 0.0)
    o = jnp.maximum(o @ w1[...] + b1[...], 0.0)
    o = jnp.maximum(o @ w2[...] + b2[...], 0.0)
    out_ref[...] = o @ w3[...] + b3[...]


def kernel(x, edge_index, edge_attr, batch, params):
    src = edge_index[0]
    dst = edge_index[1]
    ea = jax.nn.relu(_apply(params["edge_enc"][0], edge_attr))
    ea = jax.nn.relu(_apply(params["edge_enc"][1], ea))
    ea = _apply(params["edge_enc"][2], ea)
    h = jax.nn.relu(_apply(params["node_enc"][0], x))
    h = _apply(params["node_enc"][1], h)
    g = params["gine"]
    tc = params["tc"]
    sk = params["skip"]
    h0 = h
    h = _bn(_leaky(_gine(g[0], h, src, dst, ea)))
    h = _bn(_leaky(_gine(g[1], h, src, dst, ea)))
    h = _bn(_leaky(_gine(g[2], h, src, dst, ea)))
    h = _bn(h + _apply(sk["s2"], h0))
    h = _bn(_leaky(_tconv(tc[0], h, src, dst, ea)))
    h0 = h
    h = _bn(_leaky(_gine(g[3], h, src, dst, ea)))
    h = _bn(_leaky(_gine(g[4], h, src, dst, ea)))
    h = _bn(_leaky(_gine(g[5], h, src, dst, ea)))
    h = _bn(h + _apply(sk["s3"], h0))
    h = _bn(_leaky(_tconv(tc[1], h, src, dst, ea)))
    h0 = h
    h = _bn(_leaky(_gine(g[6], h, src, dst, ea)))
    h = _bn(_leaky(_gine(g[7], h, src, dst, ea)))
    h = _leaky(_gine(g[8], h, src, dst, ea))
    h = _bn(h + _apply(sk["s4"], h0))
    h0 = h
    h = _bn(_leaky(_gine(g[9], h, src, dst, ea)))
    h = _bn(_leaky(_gine(g[10], h, src, dst, ea)))
    h = _bn(h + _apply(sk["s5"], h0))
    h = _bn(_leaky(_tconv(tc[2], h, src, dst, ea)))
    ones = jnp.ones((N_NODES,), jnp.float32)
    cnt = jax.ops.segment_sum(ones, batch, num_segments=N_GRAPHS)
    pooled = jax.ops.segment_sum(h, batch, num_segments=N_GRAPHS) / jnp.maximum(cnt, 1.0)[:, None]
    fc = params["fc"]
    out = pl.pallas_call(
        _fc_kernel,
        out_shape=jax.ShapeDtypeStruct((N_GRAPHS, 64), jnp.float32),
    )(pooled, fc[0][0], fc[0][1], fc[1][0], fc[1][1], fc[2][0], fc[2][1],
      fc[3][0], fc[3][1])
    return out


# speedup vs baseline: 8.5610x; 8.5610x over previous
import functools

import jax
import jax.numpy as jnp
import numpy as np
from jax import lax
from jax.experimental import pallas as pl
from jax.experimental.pallas import tpu as pltpu
from jax.experimental.pallas import tpu_sc as plsc

N = 10000
E = 320000
D = 128
EDIM = 16
HEADS = 8
HD = 16
G = 8
NC = 2
NS = 16
NW = NC * NS
CH = 128
E_PAD = 323584
EPW = E_PAD // NW
NCH_U = EPW // CH
EB = 1024
NEB = E_PAD // EB
HALF = N // 2
N_SP = 5120
TRASH = HALF
TRASH_T = 320
NEG = -3.4e38

_MESH_CACHE = []


def _mesh():
    if not _MESH_CACHE:
        _MESH_CACHE.append(
            plsc.VectorSubcoreMesh(core_axis_name="c", subcore_axis_name="s",
                                   num_cores=NC, num_subcores=NS))
    return _MESH_CACHE[0]


def _wid():
    return lax.axis_index("c") * NS + lax.axis_index("s")


def _tile_nodes(w):
    return pl.multiple_of(((w * 625) >> 4) << 3, 8)


def _gine_edge_body(h_hbm, ea_hbm, src_hbm, dloc_hbm, est_hbm, out_hbm,
                    estv, sidx, didx, hrows, erows, agg, sema, semb):
    c = lax.axis_index("c")
    w = _wid()
    n0 = _tile_nodes(w)
    loc0 = pl.multiple_of(n0 - c * HALF, 8)
    pltpu.sync_copy(est_hbm, estv)
    ev = estv[pl.ds(pl.multiple_of(w * 8, 8), 16)]
    es = ev[0]
    ee = ev[1]
    astart = jnp.bitwise_and(es, jnp.int32(-8))
    nch = (ee - astart + (CH - 1)) // CH

    zv = jnp.zeros((16,), jnp.float32)

    def zb(i, _):
        for j in range(8):
            erows[0, i, pl.ds(j * 16, 16)] = zv
        return 0

    lax.fori_loop(0, CH, zb, 0, unroll=8)
    for r in range(2):
        pltpu.sync_copy(erows.at[0], agg.at[pl.ds(loc0 + r * CH, CH)])
    pltpu.sync_copy(erows.at[0, pl.ds(0, 64)], agg.at[pl.ds(loc0 + 2 * CH, 64)])
    plsc.subcore_barrier()

    lanes = lax.iota(jnp.int32, 16)

    def cbase(ci):
        return pl.multiple_of(astart + ci * CH, 8)

    def stage_a(ci, par):
        base = cbase(ci)
        pltpu.async_copy(src_hbm.at[pl.ds(base, CH)], sidx.at[par], sema.at[par])
        pltpu.async_copy(dloc_hbm.at[pl.ds(base, CH)], didx.at[par], sema.at[par])

    def stage_b(ci, par):
        base = cbase(ci)
        pltpu.make_async_copy(src_hbm.at[pl.ds(base, CH)], sidx.at[par],
                              sema.at[par]).wait()
        pltpu.make_async_copy(dloc_hbm.at[pl.ds(base, CH)], didx.at[par],
                              sema.at[par]).wait()
        pltpu.async_copy(h_hbm.at[sidx.at[par]], hrows.at[par], semb.at[par])
        pltpu.async_copy(ea_hbm.at[pl.ds(base, CH)], erows.at[par], semb.at[par])

        def mb(v, _):
            gid = base + v * 16 + lanes
            dv = didx[par, pl.ds(v * 16, 16)]
            ok = (gid >= es) & (gid < ee)
            didx[par, pl.ds(v * 16, 16)] = jnp.where(ok, dv, jnp.int32(TRASH))
            return 0

        lax.fori_loop(0, 8, mb, 0, unroll=8)

    @pl.when(nch > 0)
    def _():
        stage_a(0, 0)
        stage_b(0, 0)

    @pl.when(nch > 1)
    def _():
        stage_a(1, 1)

    def chunk(ci, _):
        par = lax.rem(ci, 2)
        base = cbase(ci)
        pltpu.make_async_copy(h_hbm.at[sidx.at[par]], hrows.at[par],
                              semb.at[par]).wait()
        pltpu.make_async_copy(ea_hbm.at[pl.ds(base, CH)], erows.at[par],
                              semb.at[par]).wait()

        def rowb(r, _):
            for j in range(8):
                hv = hrows[par, r, pl.ds(j * 16, 16)]
                ev2 = erows[par, r, pl.ds(j * 16, 16)]
                hrows[par, r, pl.ds(j * 16, 16)] = jnp.maximum(hv + ev2, 0.0)
            return 0

        lax.fori_loop(0, CH, rowb, 0, unroll=4)
        pltpu.sync_copy(hrows.at[par], agg.at[didx.at[par]], add=True)

        @pl.when(ci + 2 < nch)
        def _():
            stage_a(ci + 2, par)

        @pl.when(ci + 1 < nch)
        def _():
            stage_b(ci + 1, 1 - par)

        return 0

    lax.fori_loop(0, nch, chunk, 0)
    plsc.subcore_barrier()
    n1 = _tile_nodes(w + 1)
    pltpu.sync_copy(agg.at[pl.ds(loc0, 312)],
                    out_hbm.at[pl.ds(pl.multiple_of(n0, 8), 312)])

    @pl.when(n1 - n0 == 320)
    def _():
        pltpu.sync_copy(agg.at[pl.ds(loc0 + 312, 8)],
                        out_hbm.at[pl.ds(pl.multiple_of(n0 + 312, 8), 8)])


def _gine_edge(h, ea, src_p, dloc_p, est_p):
    return pl.kernel(
        _gine_edge_body,
        out_type=jax.ShapeDtypeStruct((N, D), jnp.float32),
        mesh=_mesh(),
        scratch_types=[
            pltpu.VMEM((320,), jnp.int32),
            pltpu.VMEM((2, CH), jnp.int32),
            pltpu.VMEM((2, CH), jnp.int32),
            pltpu.VMEM((2, CH, D), jnp.float32),
            pltpu.VMEM((2, CH, D), jnp.float32),
            pltpu.VMEM_SHARED((N_SP, D), jnp.float32),
            pltpu.SemaphoreType.DMA((2,)),
            pltpu.SemaphoreType.DMA((2,)),
        ],
    )(h, ea, src_p, dloc_p, est_p)


def _tgather_body(q_hbm, k_hbm, v_hbm, src_hbm, dg_hbm,
                  qd_hbm, ks_hbm, vs_hbm,
                  sidx, didx, qb, kb, vb, sema, semg, semw):
    w = _wid()
    base0 = w * EPW

    def cbase(ci):
        return pl.multiple_of(base0 + ci * CH, 8)

    def stage_a(ci, par):
        base = cbase(ci)
        pltpu.async_copy(src_hbm.at[pl.ds(base, CH)], sidx.at[par], sema.at[par])
        pltpu.async_copy(dg_hbm.at[pl.ds(base, CH)], didx.at[par], sema.at[par])

    def stage_b(ci, par):
        base = cbase(ci)
        pltpu.make_async_copy(src_hbm.at[pl.ds(base, CH)], sidx.at[par],
                              sema.at[par]).wait()
        pltpu.make_async_copy(dg_hbm.at[pl.ds(base, CH)], didx.at[par],
                              sema.at[par]).wait()
        pltpu.async_copy(q_hbm.at[didx.at[par]], qb.at[par], semg.at[par])
        pltpu.async_copy(k_hbm.at[sidx.at[par]], kb.at[par], semg.at[par])
        pltpu.async_copy(v_hbm.at[sidx.at[par]], vb.at[par], semg.at[par])

    def wait_writes(ci, par):
        base = cbase(ci)
        pltpu.make_async_copy(qb.at[par], qd_hbm.at[pl.ds(base, CH)],
                              semw.at[par]).wait()
        pltpu.make_async_copy(kb.at[par], ks_hbm.at[pl.ds(base, CH)],
                              semw.at[par]).wait()
        pltpu.make_async_copy(vb.at[par], vs_hbm.at[pl.ds(base, CH)],
                              semw.at[par]).wait()

    stage_a(0, 0)
    stage_b(0, 0)
    stage_a(1, 1)

    def chunk(ci, _):
        par = lax.rem(ci, 2)
        base = cbase(ci)
        pltpu.make_async_copy(q_hbm.at[didx.at[par]], qb.at[par],
                              semg.at[par]).wait()
        pltpu.make_async_copy(k_hbm.at[sidx.at[par]], kb.at[par],
                              semg.at[par]).wait()
        pltpu.make_async_copy(v_hbm.at[sidx.at[par]], vb.at[par],
                              semg.at[par]).wait()
        pltpu.async_copy(qb.at[par], qd_hbm.at[pl.ds(base, CH)], semw.at[par])
        pltpu.async_copy(kb.at[par], ks_hbm.at[pl.ds(base, CH)], semw.at[par])
        pltpu.async_copy(vb.at[par], vs_hbm.at[pl.ds(base, CH)], semw.at[par])

        @pl.when(ci + 2 < NCH_U)
        def _():
            stage_a(ci + 2, par)

        @pl.when(ci + 1 < NCH_U)
        def _():
            @pl.when(ci >= 1)
            def _():
                wait_writes(ci - 1, 1 - par)

            stage_b(ci + 1, 1 - par)

        return 0

    lax.fori_loop(0, NCH_U, chunk, 0)
    wait_writes(NCH_U - 2, NCH_U % 2)
    wait_writes(NCH_U - 1, (NCH_U - 1) % 2)


def _tgather(q, k, v, src_p, dg_p):
    sh = jax.ShapeDtypeStruct((E_PAD, D), jnp.float32)
    return pl.kernel(
        _tgather_body,
        out_type=(sh, sh, sh),
        mesh=_mesh(),
        scratch_types=[
            pltpu.VMEM((2, CH), jnp.int32),
            pltpu.VMEM((2, CH), jnp.int32),
            pltpu.VMEM((2, CH, D), jnp.float32),
            pltpu.VMEM((2, CH, D), jnp.float32),
            pltpu.VMEM((2, CH, D), jnp.float32),
            pltpu.SemaphoreType.DMA((2,)),
            pltpu.SemaphoreType.DMA((2,)),
            pltpu.SemaphoreType.DMA((2,)),
        ],
    )(q, k, v, src_p, dg_p)


def _amaxden_body(alf_hbm, dg_hbm, est_hbm, amax_hbm, den_hbm,
                  estv, amx, dnt, abuf, didx, sem1):
    w = _wid()
    n0 = _tile_nodes(w)
    n1 = _tile_nodes(w + 1)
    pltpu.sync_copy(est_hbm, estv)
    ev = estv[pl.ds(pl.multiple_of(w * 8, 8), 16)]
    es = ev[0]
    ee = ev[1]
    astart = jnp.bitwise_and(es, jnp.int32(-8))
    nch = (ee - astart + (CH - 1)) // CH
    off = n0

    neg = jnp.full((16,), NEG, jnp.float32)
    zv = jnp.zeros((16,), jnp.float32)

    def ib(i, _):
        amx[pl.ds(i * 16, 16)] = neg
        dnt[pl.ds(i * 16, 16)] = zv
        return 0

    lax.fori_loop(0, TRASH_T + 1, ib, 0, unroll=8)

    def chunk1(ci, _):
        base = pl.multiple_of(astart + ci * CH, 8)
        pltpu.sync_copy(alf_hbm.at[pl.ds(pl.multiple_of(base * 16, 128), CH * 16)], abuf)
        pltpu.sync_copy(dg_hbm.at[pl.ds(base, CH)], didx)

        def grp(gi, _):
            dvec = didx[pl.ds(pl.multiple_of(gi * 16, 16), 16)]
            for l in range(16):
                i = gi * 16 + l
                gid = base + i
                ok = (gid >= es) & (gid < ee)
                tl = jnp.where(ok, dvec[l] - off, jnp.int32(TRASH_T))
                av = abuf[pl.ds(pl.multiple_of(i * 16, 16), 16)]
                ts = pl.multiple_of(tl * 16, 16)
                amx[pl.ds(ts, 16)] = jnp.maximum(amx[pl.ds(ts, 16)], av)
            return 0

        lax.fori_loop(0, CH // 16, grp, 0)
        return 0

    lax.fori_loop(0, nch, chunk1, 0)

    def chunk2(ci, _):
        base = pl.multiple_of(astart + ci * CH, 8)
        pltpu.sync_copy(alf_hbm.at[pl.ds(pl.multiple_of(base * 16, 128), CH * 16)], abuf)
        pltpu.sync_copy(dg_hbm.at[pl.ds(base, CH)], didx)

        def grp(gi, _):
            dvec = didx[pl.ds(pl.multiple_of(gi * 16, 16), 16)]
            for l in range(16):
                i = gi * 16 + l
                gid = base + i
                ok = (gid >= es) & (gid < ee)
                tl = jnp.where(ok, dvec[l] - off, jnp.int32(TRASH_T))
                av = abuf[pl.ds(pl.multiple_of(i * 16, 16), 16)]
                ts = pl.multiple_of(tl * 16, 16)
                u = jnp.exp(av - amx[pl.ds(ts, 16)])
                dnt[pl.ds(ts, 16)] = dnt[pl.ds(ts, 16)] + u
            return 0

        lax.fori_loop(0, CH // 16, grp, 0)
        return 0

    lax.fori_loop(0, nch, chunk2, 0)

    pltpu.sync_copy(amx.at[pl.ds(0, 312 * 16)],
                    amax_hbm.at[pl.ds(pl.multiple_of(n0 * 16, 128), 312 * 16)])
    pltpu.sync_copy(dnt.at[pl.ds(0, 312 * 16)],
                    den_hbm.at[pl.ds(pl.multiple_of(n0 * 16, 128), 312 * 16)])

    @pl.when(n1 - n0 == 320)
    def _():
        pltpu.sync_copy(dnt.at[pl.ds(312 * 16, 128)],
                        den_hbm.at[pl.ds(pl.multiple_of((n0 + 312) * 16, 128), 128)])

    @pl.when(n1 - n0 == 320)
    def _():
        pltpu.sync_copy(amx.at[pl.ds(312 * 16, 128)],
                        amax_hbm.at[pl.ds(pl.multiple_of((n0 + 312) * 16, 128), 128)])


def _amaxden_sc(alpha_flat, dg_p, est_p):
    sh = jax.ShapeDtypeStruct((N * 16,), jnp.float32)
    return pl.kernel(
        _amaxden_body,
        out_type=(sh, sh),
        mesh=_mesh(),
        scratch_types=[
            pltpu.VMEM((320,), jnp.int32),
            pltpu.VMEM(((TRASH_T + 1) * 16,), jnp.float32),
            pltpu.VMEM(((TRASH_T + 1) * 16,), jnp.float32),
            pltpu.VMEM((CH * 16,), jnp.float32),
            pltpu.VMEM((CH,), jnp.int32),
            pltpu.SemaphoreType.DMA,
        ],
    )(alpha_flat, dg_p, est_p)


def _alnorm_body(alf_hbm, dg_hbm, ad_hbm, out_hbm,
                 abuf, adbuf, dgidx, sem1):
    w = _wid()
    base0 = w * EPW

    def chunk(ci, _):
        base = pl.multiple_of(base0 + ci * CH, 8)
        pltpu.sync_copy(alf_hbm.at[pl.ds(pl.multiple_of(base * 16, 128), CH * 16)], abuf)
        pltpu.sync_copy(dg_hbm.at[pl.ds(base, CH)], dgidx)
        cp = pltpu.async_copy(ad_hbm.at[dgidx], adbuf, sem1)
        cp.wait()

        def vrow(i, _):
            a = abuf[pl.ds(pl.multiple_of(i * 16, 16), 16)]
            m = adbuf[i, pl.ds(0, 16)]
            dn = adbuf[i, pl.ds(16, 16)]
            abuf[pl.ds(pl.multiple_of(i * 16, 16), 16)] = (
                jnp.exp(a - m) / (dn + 1e-16))
            return 0

        lax.fori_loop(0, CH, vrow, 0, unroll=4)
        pltpu.sync_copy(abuf, out_hbm.at[pl.ds(pl.multiple_of(base * 16, 128), CH * 16)])
        return 0

    lax.fori_loop(0, NCH_U, chunk, 0)


def _alnorm(alpha_flat, dg_p, ad):
    return pl.kernel(
        _alnorm_body,
        out_type=jax.ShapeDtypeStruct((E_PAD * 16,), jnp.float32),
        mesh=_mesh(),
        scratch_types=[
            pltpu.VMEM((CH * 16,), jnp.float32),
            pltpu.VMEM((CH, D), jnp.float32),
            pltpu.VMEM((CH,), jnp.int32),
            pltpu.SemaphoreType.DMA,
        ],
    )(alpha_flat, dg_p, ad)


def _scatter_body(msg_hbm, dloc_hbm, est_hbm, out_hbm,
                  estv, didx, mrows, agg, sem1):
    c = lax.axis_index("c")
    w = _wid()
    n0 = _tile_nodes(w)
    loc0 = pl.multiple_of(n0 - c * HALF, 8)
    pltpu.sync_copy(est_hbm, estv)
    ev = estv[pl.ds(pl.multiple_of(w * 8, 8), 16)]
    es = ev[0]
    ee = ev[1]
    astart = jnp.bitwise_and(es, jnp.int32(-8))
    nch = (ee - astart + (CH - 1)) // CH

    zv = jnp.zeros((16,), jnp.float32)

    def zb(i, _):
        for j in range(8):
            mrows[i, pl.ds(j * 16, 16)] = zv
        return 0

    lax.fori_loop(0, CH, zb, 0, unroll=8)
    for r in range(2):
        pltpu.sync_copy(mrows, agg.at[pl.ds(loc0 + r * CH, CH)])
    pltpu.sync_copy(mrows.at[pl.ds(0, 64)], agg.at[pl.ds(loc0 + 2 * CH, 64)])
    plsc.subcore_barrier()

    lanes = lax.iota(jnp.int32, 16)

    def chunk(ci, _):
        base = pl.multiple_of(astart + ci * CH, 8)
        pltpu.sync_copy(dloc_hbm.at[pl.ds(base, CH)], didx)
        cp = pltpu.async_copy(msg_hbm.at[pl.ds(base, CH)], mrows, sem1)

        def mb(v, _):
            gid = base + v * 16 + lanes
            dv = didx[pl.ds(v * 16, 16)]
            ok = (gid >= es) & (gid < ee)
            didx[pl.ds(v * 16, 16)] = jnp.where(ok, dv, jnp.int32(TRASH))
            return 0

        lax.fori_loop(0, 8, mb, 0, unroll=8)
        cp.wait()
        pltpu.sync_copy(mrows, agg.at[didx], add=True)
        return 0

    lax.fori_loop(0, nch, chunk, 0)
    plsc.subcore_barrier()
    n1 = _tile_nodes(w + 1)
    pltpu.sync_copy(agg.at[pl.ds(loc0, 312)],
                    out_hbm.at[pl.ds(pl.multiple_of(n0, 8), 312)])

    @pl.when(n1 - n0 == 320)
    def _():
        pltpu.sync_copy(agg.at[pl.ds(loc0 + 312, 8)],
                        out_hbm.at[pl.ds(pl.multiple_of(n0 + 312, 8), 8)])


def _scatter_rows(msg, dloc_p, est_p):
    return pl.kernel(
        _scatter_body,
        out_type=jax.ShapeDtypeStruct((N, D), jnp.float32),
        mesh=_mesh(),
        scratch_types=[
            pltpu.VMEM((320,), jnp.int32),
            pltpu.VMEM((CH,), jnp.int32),
            pltpu.VMEM((CH, D), jnp.float32),
            pltpu.VMEM_SHARED((N_SP, D), jnp.float32),
            pltpu.SemaphoreType.DMA,
        ],
    )(msg, dloc_p, est_p)


def _bn_in(t):
    m = jnp.mean(t, axis=0, keepdims=True)
    v = jnp.mean((t - m) * (t - m), axis=0, keepdims=True)
    return (t - m) / jnp.sqrt(v + 1e-5)


def _leaky_in(t):
    return jnp.where(t >= 0.0, t, 0.2 * t)


def _edge_enc_kbody(ea_ref, w0, b0, w1, b1, w2, b2, o_ref):
    t = jnp.maximum(ea_ref[...] @ w0[...] + b0[...], 0.0)
    t = jnp.maximum(t @ w1[...] + b1[...], 0.0)
    o_ref[...] = t @ w2[...] + b2[...]


def _edge_enc(ea_attr, p):
    (w0, b0), (w1, b1), (w2, b2) = p
    full = lambda shp: pl.BlockSpec(shp, lambda i: (0,) * len(shp))
    return pl.pallas_call(
        _edge_enc_kbody,
        grid=(NEB,),
        in_specs=[pl.BlockSpec((EB, EDIM), lambda i: (i, 0)),
                  full(w0.shape), full(b0.shape), full(w1.shape),
                  full(b1.shape), full(w2.shape), full(b2.shape)],
        out_specs=pl.BlockSpec((EB, D), lambda i: (i, 0)),
        out_shape=jax.ShapeDtypeStruct((E_PAD, D), jnp.float32),
    )(ea_attr, w0, b0, w1, b1, w2, b2)


def _node_enc_kbody(x_ref, w0, b0, w1, b1, o_ref):
    t = jnp.maximum(x_ref[...] @ w0[...] + b0[...], 0.0)
    o_ref[...] = t @ w1[...] + b1[...]


def _node_enc(x, p):
    (w0, b0), (w1, b1) = p
    return pl.pallas_call(
        _node_enc_kbody,
        out_shape=jax.ShapeDtypeStruct((N, D), jnp.float32),
    )(x, w0, b0, w1, b1)


def _gine_node_kbody(h_ref, agg_ref, eps_ref, w1, b1, w2, b2, o_ref, *, with_bn):
    t = (1.0 + eps_ref[0, 0]) * h_ref[...] + agg_ref[...]
    t = jnp.maximum(t @ w1[...] + b1[...], 0.0)
    t = jnp.maximum(t @ w2[...] + b2[...], 0.0)
    o_ref[...] = _bn_in(t) if with_bn else t


def _gine_node(h, agg, gp, with_bn=True):
    eps = jnp.reshape(gp["eps"], (1, 1)).astype(jnp.float32)
    (w1, b1) = gp["l1"]
    (w2, b2) = gp["l2"]
    return pl.pallas_call(
        functools.partial(_gine_node_kbody, with_bn=with_bn),
        out_shape=jax.ShapeDtypeStruct((N, D), jnp.float32),
    )(h, agg, eps, w1, b1, w2, b2)


def _skip_bn_kbody(h_ref, h0_ref, w, b, o_ref):
    o_ref[...] = _bn_in(h_ref[...] + h0_ref[...] @ w[...] + b[...])


def _skip_bn(h, h0, p):
    w, b = p
    return pl.pallas_call(
        _skip_bn_kbody,
        out_shape=jax.ShapeDtypeStruct((N, D), jnp.float32),
    )(h, h0, w, b)


def _qkv_kbody(h_ref, wq, bq, wk, bk, wv, bv, q_ref, k_ref, v_ref):
    h = h_ref[...]
    q_ref[...] = h @ wq[...] + bq[...]
    k_ref[...] = h @ wk[...] + bk[...]
    v_ref[...] = h @ wv[...] + bv[...]


def _qkv(h, tp):
    sh = jax.ShapeDtypeStruct((N, D), jnp.float32)
    return pl.pallas_call(
        _qkv_kbody,
        out_shape=(sh, sh, sh),
    )(h, tp["q"][0], tp["q"][1], tp["k"][0], tp["k"][1], tp["v"][0], tp["v"][1])


def _eproj_kbody(ea_ref, we, o_ref):
    o_ref[...] = ea_ref[...] @ we[...]


def _eproj(ea, we):
    return pl.pallas_call(
        _eproj_kbody,
        grid=(NEB,),
        in_specs=[pl.BlockSpec((EB, D), lambda i: (i, 0)),
                  pl.BlockSpec((D, D), lambda i: (0, 0))],
        out_specs=pl.BlockSpec((EB, D), lambda i: (i, 0)),
        out_shape=jax.ShapeDtypeStruct((E_PAD, D), jnp.float32),
    )(ea, we)


_SEL = np.zeros((D, 16), np.float32)
for _h in range(HEADS):
    _SEL[_h * HD:(_h + 1) * HD, _h] = 1.0
_REP = np.zeros((16, D), np.float32)
for _h in range(HEADS):
    _REP[_h, _h * HD:(_h + 1) * HD] = 1.0


def _alpha_kbody(qd_ref, ks_ref, e_ref, sel_ref, o_ref):
    t = qd_ref[...] * (ks_ref[...] + e_ref[...])
    o_ref[...] = (t @ sel_ref[...]) * (1.0 / np.sqrt(HD))


def _alpha_tc(qd, ks, e):
    return pl.pallas_call(
        _alpha_kbody,
        grid=(NEB,),
        in_specs=[pl.BlockSpec((EB, D), lambda i: (i, 0))] * 3
        + [pl.BlockSpec((D, 16), lambda i: (0, 0))],
        out_specs=pl.BlockSpec((EB, 16), lambda i: (i, 0)),
        out_shape=jax.ShapeDtypeStruct((E_PAD, 16), jnp.float32),
    )(qd, ks, e, jnp.asarray(_SEL))


def _msg_kbody(vs_ref, e_ref, al_ref, rep_ref, o_ref):
    o_ref[...] = (vs_ref[...] + e_ref[...]) * (al_ref[...] @ rep_ref[...])


def _msg_tc(vs, e, al):
    return pl.pallas_call(
        _msg_kbody,
        grid=(NEB,),
        in_specs=[pl.BlockSpec((EB, D), lambda i: (i, 0)),
                  pl.BlockSpec((EB, D), lambda i: (i, 0)),
                  pl.BlockSpec((EB, 16), lambda i: (i, 0)),
                  pl.BlockSpec((16, D), lambda i: (0, 0))],
        out_specs=pl.BlockSpec((EB, D), lambda i: (i, 0)),
        out_shape=jax.ShapeDtypeStruct((E_PAD, D), jnp.float32),
    )(vs, e, al, jnp.asarray(_REP))


def _tconv_out_kbody(magg_ref, h_ref, w, b, o_ref):
    o_ref[...] = _bn_in(_leaky_in(magg_ref[...] + h_ref[...] @ w[...] + b[...]))


def _tconv_out(magg, h, p):
    w, b = p
    return pl.pallas_call(
        _tconv_out_kbody,
        out_shape=jax.ShapeDtypeStruct((N, D), jnp.float32),
    )(magg, h, w, b)


def _pool_fc_kbody(h_ref, batch_ref, f0w, f0b, f1w, f1b, f2w, f2b, f3w, f3b,
                   o_ref):
    h = h_ref[...]
    b = batch_ref[...]
    rows = []
    cnts = []
    for g in range(G):
        mask = (b == g).astype(jnp.float32)
        rows.append(jnp.sum(h * mask, axis=0, keepdims=True))
        cnts.append(jnp.sum(mask))
    pooled = jnp.concatenate(rows, axis=0)
    cnt = jnp.stack(cnts).reshape(G, 1)
    pooled = pooled / jnp.maximum(cnt, 1.0)
    o = jnp.maximum(pooled @ f0w[...] + f0b[...], 0.0)
    o = jnp.maximum(o @ f1w[...] + f1b[...], 0.0)
    o = jnp.maximum(o @ f2w[...] + f2b[...], 0.0)
    o_ref[...] = o @ f3w[...] + f3b[...]


def _pool_fc(h, batch2d, fc):
    return pl.pallas_call(
        _pool_fc_kbody,
        out_shape=jax.ShapeDtypeStruct((G, 64), jnp.float32),
    )(h, batch2d, fc[0][0], fc[0][1], fc[1][0], fc[1][1], fc[2][0], fc[2][1],
      fc[3][0], fc[3][1])


def _gine_layer(h, ea, src_p, dloc_p, est_p, gp, with_bn=True):
    agg = _gine_edge(h, ea, src_p, dloc_p, est_p)
    return _gine_node(h, agg, gp, with_bn=with_bn)


def _tconv_layer(h, ea, src_p, dloc_p, dg_p, est_p, tp):
    q, k, v = _qkv(h, tp)
    e = _eproj(ea, tp["We"])
    qd, ks, vs = _tgather(q, k, v, src_p, dg_p)
    alpha = _alpha_tc(qd, ks, e)
    alpha_flat = jnp.reshape(alpha, (-1,))
    amax_flat, den_flat = _amaxden_sc(alpha_flat, dg_p, est_p)
    ad = jnp.concatenate(
        [jnp.reshape(amax_flat, (N, 16)), jnp.reshape(den_flat, (N, 16)),
         jnp.zeros((N, 96), jnp.float32)], axis=1)
    al_flat = _alnorm(alpha_flat, dg_p, ad)
    al = jnp.reshape(al_flat, (E_PAD, 16))
    msg = _msg_tc(vs, e, al)
    magg = _scatter_rows(msg, dloc_p, est_p)
    return _tconv_out(magg, h, tp["s"])


def kernel(x, edge_index, edge_attr, batch, params):
    src = edge_index[0].astype(jnp.int32)
    dst = edge_index[1].astype(jnp.int32)
    perm = jnp.argsort(dst)
    src_s = src[perm]
    dst_s = dst[perm]
    ea_in = edge_attr[perm]

    pad = E_PAD - E
    src_p = jnp.concatenate([src_s, jnp.zeros((pad,), jnp.int32)])
    dloc = jnp.where(dst_s >= HALF, dst_s - HALF, dst_s)
    dloc_p = jnp.concatenate([dloc, jnp.full((pad,), TRASH, jnp.int32)])
    dg_p = jnp.concatenate([dst_s, jnp.zeros((pad,), jnp.int32)])
    ea_attr_p = jnp.concatenate([ea_in, jnp.zeros((pad, EDIM), jnp.float32)])

    nstart = jnp.array([(w * N) // NW for w in range(NW + 1)], jnp.int32)
    est = jnp.searchsorted(dst_s, nstart, side="left").astype(jnp.int32)
    est_pairs = jnp.stack([est[:NW], est[1:NW + 1]], axis=1)
    est_p = jnp.zeros((320,), jnp.int32).at[
        (jnp.arange(NW) * 8)[:, None] + jnp.arange(2)[None, :]
    ].set(est_pairs)

    ea = _edge_enc(ea_attr_p, params["edge_enc"])
    h = _node_enc(x, params["node_enc"])

    g = params["gine"]
    tc = params["tc"]
    sk = params["skip"]

    args = (ea, src_p, dloc_p, est_p)
    targs = (ea, src_p, dloc_p, dg_p, est_p)

    h0 = h
    h = _gine_layer(h, *args, g[0])
    h = _gine_layer(h, *args, g[1])
    h = _gine_layer(h, *args, g[2])
    h = _skip_bn(h, h0, sk["s2"])
    h = _tconv_layer(h, *targs, tc[0])
    h0 = h
    h = _gine_layer(h, *args, g[3])
    h = _gine_layer(h, *args, g[4])
    h = _gine_layer(h, *args, g[5])
    h = _skip_bn(h, h0, sk["s3"])
    h = _tconv_layer(h, *targs, tc[1])
    h0 = h
    h = _gine_layer(h, *args, g[6])
    h = _gine_layer(h, *args, g[7])
    h = _gine_layer(h, *args, g[8], with_bn=False)
    h = _skip_bn(h, h0, sk["s4"])
    h0 = h
    h = _gine_layer(h, *args, g[9])
    h = _gine_layer(h, *args, g[10])
    h = _skip_bn(h, h0, sk["s5"])
    h = _tconv_layer(h, *targs, tc[2])

    batch2d = jnp.reshape(batch.astype(jnp.int32), (N, 1))
    return _pool_fc(h, batch2d, params["fc"])


# speedup vs baseline: 8.8123x; 1.0293x over previous
import functools

import jax
import jax.numpy as jnp
import numpy as np
from jax import lax
from jax.experimental import pallas as pl
from jax.experimental.pallas import tpu as pltpu
from jax.experimental.pallas import tpu_sc as plsc

N = 10000
E = 320000
D = 128
EDIM = 16
HEADS = 8
HD = 16
G = 8
NC = 2
NS = 16
NW = NC * NS
CH = 128
E_PAD = 323584
EPW = E_PAD // NW
NCH_U = EPW // CH
EB = 1024
NEB = E_PAD // EB
HALF = N // 2
N_SP = 5120
TRASH = HALF
TRASH_T = 320
NEG = -3.4e38

_MESH_CACHE = []


def _mesh():
    if not _MESH_CACHE:
        _MESH_CACHE.append(
            plsc.VectorSubcoreMesh(core_axis_name="c", subcore_axis_name="s",
                                   num_cores=NC, num_subcores=NS))
    return _MESH_CACHE[0]


def _wid():
    return lax.axis_index("c") * NS + lax.axis_index("s")


def _tile_nodes(w):
    return pl.multiple_of(((w * 625) >> 4) << 3, 8)


def _gine_edge_body(h_hbm, ea_hbm, src_hbm, dloc_hbm, est_hbm, out_hbm,
                    estv, sidx, didx, hrows, erows, agg, sema, semb):
    c = lax.axis_index("c")
    w = _wid()
    n0 = _tile_nodes(w)
    loc0 = pl.multiple_of(n0 - c * HALF, 8)
    pltpu.sync_copy(est_hbm, estv)
    ev = estv[pl.ds(pl.multiple_of(w * 8, 8), 16)]
    es = ev[0]
    ee = ev[1]
    astart = jnp.bitwise_and(es, jnp.int32(-8))
    nch = (ee - astart + (CH - 1)) // CH

    zv = jnp.zeros((16,), jnp.float32)

    def zb(i, _):
        for j in range(8):
            erows[0, i, pl.ds(j * 16, 16)] = zv
        return 0

    lax.fori_loop(0, CH, zb, 0, unroll=8)
    for r in range(2):
        pltpu.sync_copy(erows.at[0], agg.at[pl.ds(loc0 + r * CH, CH)])
    pltpu.sync_copy(erows.at[0, pl.ds(0, 64)], agg.at[pl.ds(loc0 + 2 * CH, 64)])
    plsc.subcore_barrier()

    lanes = lax.iota(jnp.int32, 16)

    def cbase(ci):
        return pl.multiple_of(astart + ci * CH, 8)

    def stage_a(ci, par):
        base = cbase(ci)
        pltpu.async_copy(src_hbm.at[pl.ds(base, CH)], sidx.at[par], sema.at[par])
        pltpu.async_copy(dloc_hbm.at[pl.ds(base, CH)], didx.at[par], sema.at[par])

    def stage_b(ci, par):
        base = cbase(ci)
        pltpu.make_async_copy(src_hbm.at[pl.ds(base, CH)], sidx.at[par],
                              sema.at[par]).wait()
        pltpu.make_async_copy(dloc_hbm.at[pl.ds(base, CH)], didx.at[par],
                              sema.at[par]).wait()
        pltpu.async_copy(h_hbm.at[sidx.at[par]], hrows.at[par], semb.at[par])
        pltpu.async_copy(ea_hbm.at[pl.ds(base, CH)], erows.at[par], semb.at[par])

        def mb(v, _):
            gid = base + v * 16 + lanes
            dv = didx[par, pl.ds(v * 16, 16)]
            ok = (gid >= es) & (gid < ee)
            didx[par, pl.ds(v * 16, 16)] = jnp.where(ok, dv, jnp.int32(TRASH))
            return 0

        lax.fori_loop(0, 8, mb, 0, unroll=8)

    @pl.when(nch > 0)
    def _():
        stage_a(0, 0)
        stage_b(0, 0)

    @pl.when(nch > 1)
    def _():
        stage_a(1, 1)

    def chunk(ci, _):
        par = lax.rem(ci, 2)
        base = cbase(ci)
        pltpu.make_async_copy(h_hbm.at[sidx.at[par]], hrows.at[par],
                              semb.at[par]).wait()
        pltpu.make_async_copy(ea_hbm.at[pl.ds(base, CH)], erows.at[par],
                              semb.at[par]).wait()

        def rowb(r, _):
            for j in range(8):
                hv = hrows[par, r, pl.ds(j * 16, 16)]
                ev2 = erows[par, r, pl.ds(j * 16, 16)]
                hrows[par, r, pl.ds(j * 16, 16)] = jnp.maximum(hv + ev2, 0.0)
            return 0

        lax.fori_loop(0, CH, rowb, 0, unroll=4)
        pltpu.sync_copy(hrows.at[par], agg.at[didx.at[par]], add=True)

        @pl.when(ci + 2 < nch)
        def _():
            stage_a(ci + 2, par)

        @pl.when(ci + 1 < nch)
        def _():
            stage_b(ci + 1, 1 - par)

        return 0

    lax.fori_loop(0, nch, chunk, 0)
    plsc.subcore_barrier()
    n1 = _tile_nodes(w + 1)
    pltpu.sync_copy(agg.at[pl.ds(loc0, 312)],
                    out_hbm.at[pl.ds(pl.multiple_of(n0, 8), 312)])

    @pl.when(n1 - n0 == 320)
    def _():
        pltpu.sync_copy(agg.at[pl.ds(loc0 + 312, 8)],
                        out_hbm.at[pl.ds(pl.multiple_of(n0 + 312, 8), 8)])


def _gine_edge(h, ea, src_p, dloc_p, est_p):
    return pl.kernel(
        _gine_edge_body,
        out_type=jax.ShapeDtypeStruct((N, D), jnp.float32),
        mesh=_mesh(),
        scratch_types=[
            pltpu.VMEM((320,), jnp.int32),
            pltpu.VMEM((2, CH), jnp.int32),
            pltpu.VMEM((2, CH), jnp.int32),
            pltpu.VMEM((2, CH, D), jnp.float32),
            pltpu.VMEM((2, CH, D), jnp.float32),
            pltpu.VMEM_SHARED((N_SP, D), jnp.float32),
            pltpu.SemaphoreType.DMA((2,)),
            pltpu.SemaphoreType.DMA((2,)),
        ],
    )(h, ea, src_p, dloc_p, est_p)


def _tgather_body(q_hbm, k_hbm, v_hbm, src_hbm, dg_hbm,
                  qd_hbm, ks_hbm, vs_hbm,
                  sidx, didx, qb, kb, vb, sema, semg, semw):
    w = _wid()
    base0 = w * EPW

    def cbase(ci):
        return pl.multiple_of(base0 + ci * CH, 8)

    def stage_a(ci, par):
        base = cbase(ci)
        pltpu.async_copy(src_hbm.at[pl.ds(base, CH)], sidx.at[par], sema.at[par])
        pltpu.async_copy(dg_hbm.at[pl.ds(base, CH)], didx.at[par], sema.at[par])

    def stage_b(ci, par):
        base = cbase(ci)
        pltpu.make_async_copy(src_hbm.at[pl.ds(base, CH)], sidx.at[par],
                              sema.at[par]).wait()
        pltpu.make_async_copy(dg_hbm.at[pl.ds(base, CH)], didx.at[par],
                              sema.at[par]).wait()
        pltpu.async_copy(q_hbm.at[didx.at[par]], qb.at[par], semg.at[par])
        pltpu.async_copy(k_hbm.at[sidx.at[par]], kb.at[par], semg.at[par])
        pltpu.async_copy(v_hbm.at[sidx.at[par]], vb.at[par], semg.at[par])

    def wait_writes(ci, par):
        base = cbase(ci)
        pltpu.make_async_copy(qb.at[par], qd_hbm.at[pl.ds(base, CH)],
                              semw.at[par]).wait()
        pltpu.make_async_copy(kb.at[par], ks_hbm.at[pl.ds(base, CH)],
                              semw.at[par]).wait()
        pltpu.make_async_copy(vb.at[par], vs_hbm.at[pl.ds(base, CH)],
                              semw.at[par]).wait()

    stage_a(0, 0)
    stage_b(0, 0)
    stage_a(1, 1)

    def chunk(ci, _):
        par = lax.rem(ci, 2)
        base = cbase(ci)
        pltpu.make_async_copy(q_hbm.at[didx.at[par]], qb.at[par],
                              semg.at[par]).wait()
        pltpu.make_async_copy(k_hbm.at[sidx.at[par]], kb.at[par],
                              semg.at[par]).wait()
        pltpu.make_async_copy(v_hbm.at[sidx.at[par]], vb.at[par],
                              semg.at[par]).wait()
        pltpu.async_copy(qb.at[par], qd_hbm.at[pl.ds(base, CH)], semw.at[par])
        pltpu.async_copy(kb.at[par], ks_hbm.at[pl.ds(base, CH)], semw.at[par])
        pltpu.async_copy(vb.at[par], vs_hbm.at[pl.ds(base, CH)], semw.at[par])

        @pl.when(ci + 2 < NCH_U)
        def _():
            stage_a(ci + 2, par)

        @pl.when(ci + 1 < NCH_U)
        def _():
            @pl.when(ci >= 1)
            def _():
                wait_writes(ci - 1, 1 - par)

            stage_b(ci + 1, 1 - par)

        return 0

    lax.fori_loop(0, NCH_U, chunk, 0)
    wait_writes(NCH_U - 2, NCH_U % 2)
    wait_writes(NCH_U - 1, (NCH_U - 1) % 2)


def _tgather(q, k, v, src_p, dg_p):
    sh = jax.ShapeDtypeStruct((E_PAD, D), jnp.float32)
    return pl.kernel(
        _tgather_body,
        out_type=(sh, sh, sh),
        mesh=_mesh(),
        scratch_types=[
            pltpu.VMEM((2, CH), jnp.int32),
            pltpu.VMEM((2, CH), jnp.int32),
            pltpu.VMEM((2, CH, D), jnp.float32),
            pltpu.VMEM((2, CH, D), jnp.float32),
            pltpu.VMEM((2, CH, D), jnp.float32),
            pltpu.SemaphoreType.DMA((2,)),
            pltpu.SemaphoreType.DMA((2,)),
            pltpu.SemaphoreType.DMA((2,)),
        ],
    )(q, k, v, src_p, dg_p)


def _amaxden_body(alf_hbm, dg_hbm, est_hbm, amax_hbm, den_hbm,
                  estv, amx, dnt, abuf, didx, sema):
    w = _wid()
    n0 = _tile_nodes(w)
    n1 = _tile_nodes(w + 1)
    pltpu.sync_copy(est_hbm, estv)
    ev = estv[pl.ds(pl.multiple_of(w * 8, 8), 16)]
    es = ev[0]
    ee = ev[1]
    astart = jnp.bitwise_and(es, jnp.int32(-8))
    nch = (ee - astart + (CH - 1)) // CH
    off = n0

    neg = jnp.full((16,), NEG, jnp.float32)
    zv = jnp.zeros((16,), jnp.float32)

    def ib(i, _):
        amx[pl.ds(i * 16, 16)] = neg
        dnt[pl.ds(i * 16, 16)] = zv
        return 0

    lax.fori_loop(0, TRASH_T + 1, ib, 0, unroll=8)

    def cbase(ci):
        return pl.multiple_of(astart + ci * CH, 8)

    def prefetch(ci, par):
        base = cbase(ci)
        pltpu.async_copy(
            alf_hbm.at[pl.ds(pl.multiple_of(base * 16, 128), CH * 16)],
            abuf.at[par], sema.at[par])
        pltpu.async_copy(dg_hbm.at[pl.ds(base, CH)], didx.at[par],
                         sema.at[par])

    def wait_pf(ci, par):
        base = cbase(ci)
        pltpu.make_async_copy(
            alf_hbm.at[pl.ds(pl.multiple_of(base * 16, 128), CH * 16)],
            abuf.at[par], sema.at[par]).wait()
        pltpu.make_async_copy(dg_hbm.at[pl.ds(base, CH)], didx.at[par],
                              sema.at[par]).wait()

    def run_pass(body_fn):
        @pl.when(nch > 0)
        def _():
            prefetch(0, 0)

        @pl.when(nch > 1)
        def _():
            prefetch(1, 1)

        def chunk(ci, _):
            par = lax.rem(ci, 2)
            base = cbase(ci)
            wait_pf(ci, par)

            def grp(gi, _):
                dvec = didx[par, pl.ds(pl.multiple_of(gi * 16, 16), 16)]
                for l in range(16):
                    i = gi * 16 + l
                    gid = base + i
                    ok = (gid >= es) & (gid < ee)
                    tl = jnp.where(ok, dvec[l] - off, jnp.int32(TRASH_T))
                    av = abuf[par, pl.ds(pl.multiple_of(i * 16, 16), 16)]
                    body_fn(tl, av)
                return 0

            lax.fori_loop(0, CH // 16, grp, 0)

            @pl.when(ci + 2 < nch)
            def _():
                prefetch(ci + 2, par)

            return 0

        lax.fori_loop(0, nch, chunk, 0)

    def upd_max(tl, av):
        ts = pl.multiple_of(tl * 16, 16)
        amx[pl.ds(ts, 16)] = jnp.maximum(amx[pl.ds(ts, 16)], av)

    def upd_den(tl, av):
        ts = pl.multiple_of(tl * 16, 16)
        u = jnp.exp(av - amx[pl.ds(ts, 16)])
        dnt[pl.ds(ts, 16)] = dnt[pl.ds(ts, 16)] + u

    run_pass(upd_max)
    run_pass(upd_den)

    pltpu.sync_copy(amx.at[pl.ds(0, 312 * 16)],
                    amax_hbm.at[pl.ds(pl.multiple_of(n0 * 16, 128), 312 * 16)])
    pltpu.sync_copy(dnt.at[pl.ds(0, 312 * 16)],
                    den_hbm.at[pl.ds(pl.multiple_of(n0 * 16, 128), 312 * 16)])

    @pl.when(n1 - n0 == 320)
    def _():
        pltpu.sync_copy(dnt.at[pl.ds(312 * 16, 128)],
                        den_hbm.at[pl.ds(pl.multiple_of((n0 + 312) * 16, 128), 128)])

    @pl.when(n1 - n0 == 320)
    def _():
        pltpu.sync_copy(amx.at[pl.ds(312 * 16, 128)],
                        amax_hbm.at[pl.ds(pl.multiple_of((n0 + 312) * 16, 128), 128)])


def _amaxden_sc(alpha_flat, dg_p, est_p):
    sh = jax.ShapeDtypeStruct((N * 16,), jnp.float32)
    return pl.kernel(
        _amaxden_body,
        out_type=(sh, sh),
        mesh=_mesh(),
        scratch_types=[
            pltpu.VMEM((320,), jnp.int32),
            pltpu.VMEM(((TRASH_T + 1) * 16,), jnp.float32),
            pltpu.VMEM(((TRASH_T + 1) * 16,), jnp.float32),
            pltpu.VMEM((2, CH * 16), jnp.float32),
            pltpu.VMEM((2, CH), jnp.int32),
            pltpu.SemaphoreType.DMA((2,)),
        ],
    )(alpha_flat, dg_p, est_p)


def _alnorm_body(alf_hbm, dg_hbm, ad_hbm, out_hbm,
                 abuf, adbuf, dgidx, sem1):
    w = _wid()
    base0 = w * EPW

    def chunk(ci, _):
        base = pl.multiple_of(base0 + ci * CH, 8)
        pltpu.sync_copy(alf_hbm.at[pl.ds(pl.multiple_of(base * 16, 128), CH * 16)], abuf)
        pltpu.sync_copy(dg_hbm.at[pl.ds(base, CH)], dgidx)
        cp = pltpu.async_copy(ad_hbm.at[dgidx], adbuf, sem1)
        cp.wait()

        def vrow(i, _):
            a = abuf[pl.ds(pl.multiple_of(i * 16, 16), 16)]
            m = adbuf[i, pl.ds(0, 16)]
            dn = adbuf[i, pl.ds(16, 16)]
            abuf[pl.ds(pl.multiple_of(i * 16, 16), 16)] = (
                jnp.exp(a - m) / (dn + 1e-16))
            return 0

        lax.fori_loop(0, CH, vrow, 0, unroll=4)
        pltpu.sync_copy(abuf, out_hbm.at[pl.ds(pl.multiple_of(base * 16, 128), CH * 16)])
        return 0

    lax.fori_loop(0, NCH_U, chunk, 0)


def _alnorm(alpha_flat, dg_p, ad):
    return pl.kernel(
        _alnorm_body,
        out_type=jax.ShapeDtypeStruct((E_PAD * 16,), jnp.float32),
        mesh=_mesh(),
        scratch_types=[
            pltpu.VMEM((CH * 16,), jnp.float32),
            pltpu.VMEM((CH, D), jnp.float32),
            pltpu.VMEM((CH,), jnp.int32),
            pltpu.SemaphoreType.DMA,
        ],
    )(alpha_flat, dg_p, ad)


def _scatter_body(msg_hbm, dloc_hbm, est_hbm, out_hbm,
                  estv, didx, mrows, agg, sem1):
    c = lax.axis_index("c")
    w = _wid()
    n0 = _tile_nodes(w)
    loc0 = pl.multiple_of(n0 - c * HALF, 8)
    pltpu.sync_copy(est_hbm, estv)
    ev = estv[pl.ds(pl.multiple_of(w * 8, 8), 16)]
    es = ev[0]
    ee = ev[1]
    astart = jnp.bitwise_and(es, jnp.int32(-8))
    nch = (ee - astart + (CH - 1)) // CH

    zv = jnp.zeros((16,), jnp.float32)

    def zb(i, _):
        for j in range(8):
            mrows[i, pl.ds(j * 16, 16)] = zv
        return 0

    lax.fori_loop(0, CH, zb, 0, unroll=8)
    for r in range(2):
        pltpu.sync_copy(mrows, agg.at[pl.ds(loc0 + r * CH, CH)])
    pltpu.sync_copy(mrows.at[pl.ds(0, 64)], agg.at[pl.ds(loc0 + 2 * CH, 64)])
    plsc.subcore_barrier()

    lanes = lax.iota(jnp.int32, 16)

    def chunk(ci, _):
        base = pl.multiple_of(astart + ci * CH, 8)
        pltpu.sync_copy(dloc_hbm.at[pl.ds(base, CH)], didx)
        cp = pltpu.async_copy(msg_hbm.at[pl.ds(base, CH)], mrows, sem1)

        def mb(v, _):
            gid = base + v * 16 + lanes
            dv = didx[pl.ds(v * 16, 16)]
            ok = (gid >= es) & (gid < ee)
            didx[pl.ds(v * 16, 16)] = jnp.where(ok, dv, jnp.int32(TRASH))
            return 0

        lax.fori_loop(0, 8, mb, 0, unroll=8)
        cp.wait()
        pltpu.sync_copy(mrows, agg.at[didx], add=True)
        return 0

    lax.fori_loop(0, nch, chunk, 0)
    plsc.subcore_barrier()
    n1 = _tile_nodes(w + 1)
    pltpu.sync_copy(agg.at[pl.ds(loc0, 312)],
                    out_hbm.at[pl.ds(pl.multiple_of(n0, 8), 312)])

    @pl.when(n1 - n0 == 320)
    def _():
        pltpu.sync_copy(agg.at[pl.ds(loc0 + 312, 8)],
                        out_hbm.at[pl.ds(pl.multiple_of(n0 + 312, 8), 8)])


def _scatter_rows(msg, dloc_p, est_p):
    return pl.kernel(
        _scatter_body,
        out_type=jax.ShapeDtypeStruct((N, D), jnp.float32),
        mesh=_mesh(),
        scratch_types=[
            pltpu.VMEM((320,), jnp.int32),
            pltpu.VMEM((CH,), jnp.int32),
            pltpu.VMEM((CH, D), jnp.float32),
            pltpu.VMEM_SHARED((N_SP, D), jnp.float32),
            pltpu.SemaphoreType.DMA,
        ],
    )(msg, dloc_p, est_p)


def _bn_in(t):
    m = jnp.mean(t, axis=0, keepdims=True)
    v = jnp.mean((t - m) * (t - m), axis=0, keepdims=True)
    return (t - m) / jnp.sqrt(v + 1e-5)


def _leaky_in(t):
    return jnp.where(t >= 0.0, t, 0.2 * t)


def _edge_enc_kbody(ea_ref, w0, b0, w1, b1, w2, b2, o_ref):
    t = jnp.maximum(ea_ref[...] @ w0[...] + b0[...], 0.0)
    t = jnp.maximum(t @ w1[...] + b1[...], 0.0)
    o_ref[...] = t @ w2[...] + b2[...]


def _edge_enc(ea_attr, p):
    (w0, b0), (w1, b1), (w2, b2) = p
    full = lambda shp: pl.BlockSpec(shp, lambda i: (0,) * len(shp))
    return pl.pallas_call(
        _edge_enc_kbody,
        grid=(NEB,),
        in_specs=[pl.BlockSpec((EB, EDIM), lambda i: (i, 0)),
                  full(w0.shape), full(b0.shape), full(w1.shape),
                  full(b1.shape), full(w2.shape), full(b2.shape)],
        out_specs=pl.BlockSpec((EB, D), lambda i: (i, 0)),
        out_shape=jax.ShapeDtypeStruct((E_PAD, D), jnp.float32),
    )(ea_attr, w0, b0, w1, b1, w2, b2)


def _node_enc_kbody(x_ref, w0, b0, w1, b1, o_ref):
    t = jnp.maximum(x_ref[...] @ w0[...] + b0[...], 0.0)
    o_ref[...] = t @ w1[...] + b1[...]


def _node_enc(x, p):
    (w0, b0), (w1, b1) = p
    return pl.pallas_call(
        _node_enc_kbody,
        out_shape=jax.ShapeDtypeStruct((N, D), jnp.float32),
    )(x, w0, b0, w1, b1)


def _gine_node_kbody(h_ref, agg_ref, eps_ref, w1, b1, w2, b2, o_ref, *, with_bn):
    t = (1.0 + eps_ref[0, 0]) * h_ref[...] + agg_ref[...]
    t = jnp.maximum(t @ w1[...] + b1[...], 0.0)
    t = jnp.maximum(t @ w2[...] + b2[...], 0.0)
    o_ref[...] = _bn_in(t) if with_bn else t


def _gine_node(h, agg, gp, with_bn=True):
    eps = jnp.reshape(gp["eps"], (1, 1)).astype(jnp.float32)
    (w1, b1) = gp["l1"]
    (w2, b2) = gp["l2"]
    return pl.pallas_call(
        functools.partial(_gine_node_kbody, with_bn=with_bn),
        out_shape=jax.ShapeDtypeStruct((N, D), jnp.float32),
    )(h, agg, eps, w1, b1, w2, b2)


def _skip_bn_kbody(h_ref, h0_ref, w, b, o_ref):
    o_ref[...] = _bn_in(h_ref[...] + h0_ref[...] @ w[...] + b[...])


def _skip_bn(h, h0, p):
    w, b = p
    return pl.pallas_call(
        _skip_bn_kbody,
        out_shape=jax.ShapeDtypeStruct((N, D), jnp.float32),
    )(h, h0, w, b)


def _qkv_kbody(h_ref, wq, bq, wk, bk, wv, bv, q_ref, k_ref, v_ref):
    h = h_ref[...]
    q_ref[...] = h @ wq[...] + bq[...]
    k_ref[...] = h @ wk[...] + bk[...]
    v_ref[...] = h @ wv[...] + bv[...]


def _qkv(h, tp):
    sh = jax.ShapeDtypeStruct((N, D), jnp.float32)
    return pl.pallas_call(
        _qkv_kbody,
        out_shape=(sh, sh, sh),
    )(h, tp["q"][0], tp["q"][1], tp["k"][0], tp["k"][1], tp["v"][0], tp["v"][1])


def _eproj_kbody(ea_ref, we, o_ref):
    o_ref[...] = ea_ref[...] @ we[...]


def _eproj(ea, we):
    return pl.pallas_call(
        _eproj_kbody,
        grid=(NEB,),
        in_specs=[pl.BlockSpec((EB, D), lambda i: (i, 0)),
                  pl.BlockSpec((D, D), lambda i: (0, 0))],
        out_specs=pl.BlockSpec((EB, D), lambda i: (i, 0)),
        out_shape=jax.ShapeDtypeStruct((E_PAD, D), jnp.float32),
    )(ea, we)


_SEL = np.zeros((D, 16), np.float32)
for _h in range(HEADS):
    _SEL[_h * HD:(_h + 1) * HD, _h] = 1.0
_REP = np.zeros((16, D), np.float32)
for _h in range(HEADS):
    _REP[_h, _h * HD:(_h + 1) * HD] = 1.0


def _alpha_kbody(qd_ref, ks_ref, e_ref, sel_ref, o_ref):
    t = qd_ref[...] * (ks_ref[...] + e_ref[...])
    o_ref[...] = (t @ sel_ref[...]) * (1.0 / np.sqrt(HD))


def _alpha_tc(qd, ks, e):
    return pl.pallas_call(
        _alpha_kbody,
        grid=(NEB,),
        in_specs=[pl.BlockSpec((EB, D), lambda i: (i, 0))] * 3
        + [pl.BlockSpec((D, 16), lambda i: (0, 0))],
        out_specs=pl.BlockSpec((EB, 16), lambda i: (i, 0)),
        out_shape=jax.ShapeDtypeStruct((E_PAD, 16), jnp.float32),
    )(qd, ks, e, jnp.asarray(_SEL))


def _msg_kbody(vs_ref, e_ref, al_ref, rep_ref, o_ref):
    o_ref[...] = (vs_ref[...] + e_ref[...]) * (al_ref[...] @ rep_ref[...])


def _msg_tc(vs, e, al):
    return pl.pallas_call(
        _msg_kbody,
        grid=(NEB,),
        in_specs=[pl.BlockSpec((EB, D), lambda i: (i, 0)),
                  pl.BlockSpec((EB, D), lambda i: (i, 0)),
                  pl.BlockSpec((EB, 16), lambda i: (i, 0)),
                  pl.BlockSpec((16, D), lambda i: (0, 0))],
        out_specs=pl.BlockSpec((EB, D), lambda i: (i, 0)),
        out_shape=jax.ShapeDtypeStruct((E_PAD, D), jnp.float32),
    )(vs, e, al, jnp.asarray(_REP))


def _tconv_out_kbody(magg_ref, h_ref, w, b, o_ref):
    o_ref[...] = _bn_in(_leaky_in(magg_ref[...] + h_ref[...] @ w[...] + b[...]))


def _tconv_out(magg, h, p):
    w, b = p
    return pl.pallas_call(
        _tconv_out_kbody,
        out_shape=jax.ShapeDtypeStruct((N, D), jnp.float32),
    )(magg, h, w, b)


def _pool_fc_kbody(h_ref, batch_ref, f0w, f0b, f1w, f1b, f2w, f2b, f3w, f3b,
                   o_ref):
    h = h_ref[...]
    b = batch_ref[...]
    rows = []
    cnts = []
    for g in range(G):
        mask = (b == g).astype(jnp.float32)
        rows.append(jnp.sum(h * mask, axis=0, keepdims=True))
        cnts.append(jnp.sum(mask))
    pooled = jnp.concatenate(rows, axis=0)
    cnt = jnp.stack(cnts).reshape(G, 1)
    pooled = pooled / jnp.maximum(cnt, 1.0)
    o = jnp.maximum(pooled @ f0w[...] + f0b[...], 0.0)
    o = jnp.maximum(o @ f1w[...] + f1b[...], 0.0)
    o = jnp.maximum(o @ f2w[...] + f2b[...], 0.0)
    o_ref[...] = o @ f3w[...] + f3b[...]


def _pool_fc(h, batch2d, fc):
    return pl.pallas_call(
        _pool_fc_kbody,
        out_shape=jax.ShapeDtypeStruct((G, 64), jnp.float32),
    )(h, batch2d, fc[0][0], fc[0][1], fc[1][0], fc[1][1], fc[2][0], fc[2][1],
      fc[3][0], fc[3][1])


def _gine_layer(h, ea, src_p, dloc_p, est_p, gp, with_bn=True):
    agg = _gine_edge(h, ea, src_p, dloc_p, est_p)
    return _gine_node(h, agg, gp, with_bn=with_bn)


def _tconv_layer(h, ea, src_p, dloc_p, dg_p, est_p, tp):
    q, k, v = _qkv(h, tp)
    e = _eproj(ea, tp["We"])
    qd, ks, vs = _tgather(q, k, v, src_p, dg_p)
    alpha = _alpha_tc(qd, ks, e)
    alpha_flat = jnp.reshape(alpha, (-1,))
    amax_flat, den_flat = _amaxden_sc(alpha_flat, dg_p, est_p)
    ad = jnp.concatenate(
        [jnp.reshape(amax_flat, (N, 16)), jnp.reshape(den_flat, (N, 16)),
         jnp.zeros((N, 96), jnp.float32)], axis=1)
    al_flat = _alnorm(alpha_flat, dg_p, ad)
    al = jnp.reshape(al_flat, (E_PAD, 16))
    msg = _msg_tc(vs, e, al)
    magg = _scatter_rows(msg, dloc_p, est_p)
    return _tconv_out(magg, h, tp["s"])


def kernel(x, edge_index, edge_attr, batch, params):
    src = edge_index[0].astype(jnp.int32)
    dst = edge_index[1].astype(jnp.int32)
    perm = jnp.argsort(dst)
    src_s = src[perm]
    dst_s = dst[perm]
    ea_in = edge_attr[perm]

    pad = E_PAD - E
    src_p = jnp.concatenate([src_s, jnp.zeros((pad,), jnp.int32)])
    dloc = jnp.where(dst_s >= HALF, dst_s - HALF, dst_s)
    dloc_p = jnp.concatenate([dloc, jnp.full((pad,), TRASH, jnp.int32)])
    dg_p = jnp.concatenate([dst_s, jnp.zeros((pad,), jnp.int32)])
    ea_attr_p = jnp.concatenate([ea_in, jnp.zeros((pad, EDIM), jnp.float32)])

    nstart = jnp.array([(w * N) // NW for w in range(NW + 1)], jnp.int32)
    est = jnp.searchsorted(dst_s, nstart, side="left").astype(jnp.int32)
    est_pairs = jnp.stack([est[:NW], est[1:NW + 1]], axis=1)
    est_p = jnp.zeros((320,), jnp.int32).at[
        (jnp.arange(NW) * 8)[:, None] + jnp.arange(2)[None, :]
    ].set(est_pairs)

    ea = _edge_enc(ea_attr_p, params["edge_enc"])
    h = _node_enc(x, params["node_enc"])

    g = params["gine"]
    tc = params["tc"]
    sk = params["skip"]

    args = (ea, src_p, dloc_p, est_p)
    targs = (ea, src_p, dloc_p, dg_p, est_p)

    h0 = h
    h = _gine_layer(h, *args, g[0])
    h = _gine_layer(h, *args, g[1])
    h = _gine_layer(h, *args, g[2])
    h = _skip_bn(h, h0, sk["s2"])
    h = _tconv_layer(h, *targs, tc[0])
    h0 = h
    h = _gine_layer(h, *args, g[3])
    h = _gine_layer(h, *args, g[4])
    h = _gine_layer(h, *args, g[5])
    h = _skip_bn(h, h0, sk["s3"])
    h = _tconv_layer(h, *targs, tc[1])
    h0 = h
    h = _gine_layer(h, *args, g[6])
    h = _gine_layer(h, *args, g[7])
    h = _gine_layer(h, *args, g[8], with_bn=False)
    h = _skip_bn(h, h0, sk["s4"])
    h0 = h
    h = _gine_layer(h, *args, g[9])
    h = _gine_layer(h, *args, g[10])
    h = _skip_bn(h, h0, sk["s5"])
    h = _tconv_layer(h, *targs, tc[2])

    batch2d = jnp.reshape(batch.astype(jnp.int32), (N, 1))
    return _pool_fc(h, batch2d, params["fc"])
